# baseline (speedup 1.0000x reference)
import functools

import jax
import jax.numpy as jnp
from jax import lax
from jax.experimental import pallas as pl
from jax.experimental.pallas import tpu as pltpu
from jax.experimental.pallas import tpu_sc as plsc

_VOCAB = 2048
_HIDDEN = 32
_BATCH = 4
_SEQ = 8192

_NUM_CORES = 2
_NUM_SUBCORES = 16
_NW = _NUM_CORES * _NUM_SUBCORES
_TOTAL = _BATCH * _SEQ
_PER_W = _TOTAL // _NW
_CHUNK = 128
_NCH = _PER_W // _CHUNK
_W_PER_B = _NW // _BATCH
_S_PER_W = _SEQ // _W_PER_B

_GROWS = _TOTAL * _HIDDEN // (8 * 128)
_G_PER_CHUNK = _CHUNK * _HIDDEN // (8 * 128)

_sc_mesh = plsc.VectorSubcoreMesh(
    core_axis_name="c", subcore_axis_name="s",
    num_cores=_NUM_CORES, num_subcores=_NUM_SUBCORES,
)


@functools.partial(
    pl.kernel,
    out_type=jax.ShapeDtypeStruct((_TOTAL, _HIDDEN), jnp.float32),
    mesh=_sc_mesh,
    scratch_types=[
        pltpu.VMEM((_PER_W,), jnp.int32),
        pltpu.VMEM((_NCH, _CHUNK, _HIDDEN), jnp.float32),
        pltpu.SemaphoreType.DMA,
        pltpu.SemaphoreType.DMA,
    ],
    compiler_params=pltpu.CompilerParams(use_tc_tiling_on_sc=False),
)
def _sc_gather(ids_hbm, table_hbm, out_hbm, idx_v, rows_v, sem_g, sem_w):
    wid = lax.axis_index("s") * _NUM_CORES + lax.axis_index("c")
    b = wid // _W_PER_B
    s0 = (wid % _W_PER_B) * _S_PER_W
    pltpu.sync_copy(ids_hbm.at[b, pl.ds(s0, _S_PER_W)], idx_v)
    gathers = [
        pltpu.async_copy(
            table_hbm.at[idx_v.at[pl.ds(j * _CHUNK, _CHUNK)]],
            rows_v.at[j], sem_g)
        for j in range(_NCH)
    ]
    base = wid * _PER_W
    writes = []
    for j in range(_NCH):
        gathers[j].wait()
        writes.append(pltpu.async_copy(
            rows_v.at[j],
            out_hbm.at[pl.ds(base + j * _CHUNK, _CHUNK)],
            sem_w))
    for w in writes:
        w.wait()


def _proj_body(g_ref, w4_ref, b4_ref, out_ref):
    out_ref[...] = jnp.dot(
        g_ref[...], w4_ref[...], preferred_element_type=jnp.float32,
    ) + b4_ref[...]


def _project_packed(g, proj_W, proj_b):
    w4 = jnp.kron(jnp.eye(4, dtype=jnp.float32), proj_W.T)
    b4 = jnp.tile(proj_b, 4)[None]
    grid = 8
    rows = _TOTAL * _HIDDEN // 128
    return pl.pallas_call(
        _proj_body,
        grid=(grid,),
        in_specs=[
            pl.BlockSpec((rows // grid, 128), lambda i: (i, 0)),
            pl.BlockSpec((128, 128), lambda i: (0, 0)),
            pl.BlockSpec((1, 128), lambda i: (0, 0)),
        ],
        out_specs=pl.BlockSpec((rows // grid, 128), lambda i: (i, 0)),
        out_shape=jax.ShapeDtypeStruct((rows, 128), jnp.float32),
    )(g, w4, b4)


def kernel(input_ids, attention_mask, return_dict, embed_table, proj_W, proj_b):
    del attention_mask, return_dict
    g = _sc_gather(input_ids.astype(jnp.int32), embed_table)
    y = _project_packed(g.reshape(_TOTAL * _HIDDEN // 128, 128), proj_W, proj_b)
    return y.reshape(_BATCH, _SEQ, _HIDDEN)


# speedup vs baseline: 1.1093x; 1.1093x over previous
import functools

import jax
import jax.numpy as jnp
from jax import lax
from jax.experimental import pallas as pl
from jax.experimental.pallas import tpu as pltpu
from jax.experimental.pallas import tpu_sc as plsc

_VOCAB = 2048
_HIDDEN = 32
_BATCH = 4
_SEQ = 8192

_NUM_CORES = 2
_NUM_SUBCORES = 16
_NW = _NUM_CORES * _NUM_SUBCORES
_TOTAL = _BATCH * _SEQ
_PER_W = _TOTAL // _NW
_CHUNK = 128
_NCH = _PER_W // _CHUNK
_W_PER_B = _NW // _BATCH
_S_PER_W = _SEQ // _W_PER_B
_LANES = 16


def _proj_body(table_ref, w_ref, b_ref, out_ref):
    out_ref[...] = lax.dot_general(
        table_ref[...], w_ref[...],
        dimension_numbers=(((1,), (1,)), ((), ())),
        preferred_element_type=jnp.float32,
    ) + b_ref[...]


def _project_table(embed_table, proj_W, proj_b):
    return pl.pallas_call(
        _proj_body,
        out_shape=jax.ShapeDtypeStruct((_VOCAB, _HIDDEN), jnp.float32),
    )(embed_table, proj_W, proj_b.reshape(1, _HIDDEN))


_sc_mesh = plsc.VectorSubcoreMesh(
    core_axis_name="c", subcore_axis_name="s",
    num_cores=_NUM_CORES, num_subcores=_NUM_SUBCORES,
)


@functools.partial(
    pl.kernel,
    out_type=jax.ShapeDtypeStruct(
        (_BATCH, _HIDDEN // 8, _SEQ // 128, 8, 128), jnp.float32),
    mesh=_sc_mesh,
    scratch_types=[
        pltpu.VMEM((_PER_W,), jnp.int32),
        pltpu.VMEM((_NCH, _CHUNK, _HIDDEN), jnp.float32),
        pltpu.VMEM((_NCH, _HIDDEN, _CHUNK), jnp.float32),
        pltpu.SemaphoreType.DMA,
        pltpu.SemaphoreType.DMA,
    ],
    compiler_params=pltpu.CompilerParams(
        use_tc_tiling_on_sc=False, needs_layout_passes=False),
)
def _sc_gather(ids_hbm, table_hbm, out_hbm, idx_v, rows_v, t_v, sem_g, sem_w):
    wid = lax.axis_index("s") * _NUM_CORES + lax.axis_index("c")
    b = wid // _W_PER_B
    j_base = (wid % _W_PER_B) * _NCH
    s0 = (wid % _W_PER_B) * _S_PER_W
    pltpu.sync_copy(ids_hbm.at[b, pl.ds(s0, _S_PER_W)], idx_v)
    gathers = [
        pltpu.async_copy(
            table_hbm.at[idx_v.at[pl.ds(j * _CHUNK, _CHUNK)]],
            rows_v.at[j], sem_g)
        for j in range(_NCH)
    ]
    lane_iota = lax.iota(jnp.int32, _LANES)
    writes = []
    for j in range(_NCH):
        gathers[j].wait()

        def _row(h, _, j=j):
            for g in range(_CHUNK // _LANES):
                vec = plsc.load_gather(
                    rows_v.at[j],
                    [lane_iota + (g * _LANES), jnp.full((_LANES,), h, jnp.int32)],
                )
                t_v[j, h, pl.ds(g * _LANES, _LANES)] = vec
            return _

        lax.fori_loop(0, _HIDDEN, _row, 0, unroll=False)

        for i in range(_HIDDEN // 8):
            writes.append(pltpu.async_copy(
                t_v.at[j, pl.ds(8 * i, 8)],
                out_hbm.at[b, i, j_base + j],
                sem_w))
    for w in writes:
        w.wait()


def kernel(input_ids, attention_mask, return_dict, embed_table, proj_W, proj_b):
    del attention_mask, return_dict
    projected = _project_table(embed_table, proj_W, proj_b)
    out5 = _sc_gather(input_ids.astype(jnp.int32), projected)
    return out5.transpose(0, 2, 4, 1, 3).reshape(_BATCH, _SEQ, _HIDDEN)
